# baseline (speedup 1.0000x reference)
import functools

import jax
import jax.numpy as jnp
from jax import lax
from jax.experimental import pallas as pl
from jax.experimental.pallas import tpu as pltpu

_EPS = 1e-5
_SLOPE = 0.1


def _slot_dot(w_ref, im_ref, d, C, Mp):
    r = d % 4
    K = 9 * C
    if r <= 1:
        im = im_ref[r:r + 3].reshape(27 * C, Mp)
        return jnp.dot(w_ref[...], im, preferred_element_type=jnp.float32)
    if r == 2:
        im_a = im_ref[2:4].reshape(18 * C, Mp)
        im_b = im_ref[0]
        ya = jnp.dot(w_ref[:, :2 * K], im_a,
                     preferred_element_type=jnp.float32)
        yb = jnp.dot(w_ref[:, 2 * K:], im_b,
                     preferred_element_type=jnp.float32)
        return ya + yb
    im_a = im_ref[3]
    im_b = im_ref[0:2].reshape(18 * C, Mp)
    ya = jnp.dot(w_ref[:, :K], im_a, preferred_element_type=jnp.float32)
    yb = jnp.dot(w_ref[:, K:], im_b, preferred_element_type=jnp.float32)
    return ya + yb


def _conv1_kernel(x_ref, w_ref, mask_ref, y_ref, stats_ref, im_ref,
                  *, Cin, D, H, W):
    Wp = W + 2
    Mp = H * Wp

    def build(p):
        plane = x_ref[p]
        slot = p % 4
        for t in range(9):
            s = (t // 3) * Wp + (t % 3)
            im_ref[slot, t * Cin:(t + 1) * Cin, :] = plane[:, s:s + Mp]

    build(0)
    build(1)
    ssum = None
    for d in range(D):
        build(d + 2)
        y = _slot_dot(w_ref, im_ref, d, Cin, Mp)
        y_ref[d] = y.astype(y_ref.dtype)
        yv = y * mask_ref[...]
        part = jnp.concatenate(
            [jnp.sum(yv, axis=1, keepdims=True),
             jnp.sum(yv * yv, axis=1, keepdims=True)], axis=1)
        ssum = part if ssum is None else ssum + part

    stats_ref[...] = ssum


def _conv2_kernel(y1_ref, w_ref, mask_ref, sc_ref, sh_ref,
                  y2_ref, stats_ref, im_ref, *, Cout, D, H, W):
    Wp = W + 2
    Mp = H * Wp

    def build(p):
        slot = p % 4
        if p < 1 or p > D:
            im_ref[slot] = jnp.zeros((9 * Cout, Mp), jnp.bfloat16)
            return
        a = y1_ref[p - 1].astype(jnp.float32)
        a = a * sc_ref[...] + sh_ref[...]
        a = jnp.where(a >= 0.0, a, _SLOPE * a)
        a = a * mask_ref[...]
        ab = a.astype(jnp.bfloat16)
        for t in range(9):
            o = Wp + 1 - ((t // 3) * Wp + (t % 3))
            if o > 0:
                z = jnp.zeros((Cout, o), jnp.bfloat16)
                tap = jnp.concatenate([z, ab[:, :Mp - o]], axis=1)
            elif o < 0:
                z = jnp.zeros((Cout, -o), jnp.bfloat16)
                tap = jnp.concatenate([ab[:, -o:], z], axis=1)
            else:
                tap = ab
            im_ref[slot, t * Cout:(t + 1) * Cout, :] = tap

    build(0)
    build(1)
    ssum = None
    for d in range(D):
        build(d + 2)
        y = _slot_dot(w_ref, im_ref, d, Cout, Mp)
        y2_ref[d] = y.astype(y2_ref.dtype)
        yv = y * mask_ref[...]
        part = jnp.concatenate(
            [jnp.sum(yv, axis=1, keepdims=True),
             jnp.sum(yv * yv, axis=1, keepdims=True)], axis=1)
        ssum = part if ssum is None else ssum + part

    stats_ref[...] = ssum


def _bn_act_kernel(y_ref, sc_ref, sh_ref, o_ref, *, H, W):
    Wp = W + 2
    z = y_ref[...].astype(jnp.float32) * sc_ref[...] + sh_ref[...]
    z = jnp.where(z >= 0.0, z, _SLOPE * z)
    Cout = z.shape[0]
    o_ref[...] = z.reshape(Cout, H, Wp)[:, :, :W]


def _fold_stats(stats, gamma, beta, count):
    total = jnp.sum(stats, axis=0)
    mean = total[:, 0] / count
    var = total[:, 1] / count - mean * mean
    scale = gamma * lax.rsqrt(var + _EPS)
    shift = beta - mean * scale
    Cout = gamma.shape[0]
    return scale.reshape(Cout, 1), shift.reshape(Cout, 1)


@jax.jit
def _forward(x, w1, g1, be1, w2, g2, be2):
    N, Cin, D, H, W = x.shape
    Cout = w1.shape[-1]
    Wp = W + 2
    Mp = H * Wp
    Lp = (H + 3) * Wp

    xp = jnp.pad(jnp.transpose(x, (0, 2, 1, 3, 4)),
                 ((0, 0), (1, 1), (0, 0), (1, 2), (1, 1)))
    xf = xp.reshape(N, D + 2, Cin, Lp).astype(jnp.bfloat16)

    w1m = jnp.transpose(w1, (4, 0, 1, 2, 3)).reshape(Cout, 27 * Cin)
    w1m = w1m.astype(jnp.bfloat16)
    w2m = jnp.transpose(w2, (4, 0, 1, 2, 3)).reshape(Cout, 27 * Cout)
    w2m = w2m.astype(jnp.bfloat16)

    mask = ((jnp.arange(Mp, dtype=jnp.int32) % Wp) < W)
    mask = mask.astype(jnp.float32).reshape(1, Mp)

    k1 = functools.partial(_conv1_kernel, Cin=Cin, D=D, H=H, W=W)
    y1, stats1 = pl.pallas_call(
        k1,
        out_shape=(jax.ShapeDtypeStruct((N, D, Cout, Mp), jnp.bfloat16),
                   jax.ShapeDtypeStruct((N, Cout, 2), jnp.float32)),
        grid_spec=pltpu.PrefetchScalarGridSpec(
            num_scalar_prefetch=0,
            grid=(N,),
            in_specs=[
                pl.BlockSpec((None, D + 2, Cin, Lp), lambda n: (n, 0, 0, 0)),
                pl.BlockSpec((Cout, 27 * Cin), lambda n: (0, 0)),
                pl.BlockSpec((1, Mp), lambda n: (0, 0)),
            ],
            out_specs=[
                pl.BlockSpec((None, D, Cout, Mp), lambda n: (n, 0, 0, 0)),
                pl.BlockSpec((None, Cout, 2), lambda n: (n, 0, 0)),
            ],
            scratch_shapes=[pltpu.VMEM((4, 9 * Cin, Mp), jnp.bfloat16)],
        ),
        compiler_params=pltpu.CompilerParams(
            dimension_semantics=("parallel",)),
    )(xf, w1m, mask)

    count = jnp.float32(N * D * H * W)
    sc1, sh1 = _fold_stats(stats1, g1, be1, count)

    k2 = functools.partial(_conv2_kernel, Cout=Cout, D=D, H=H, W=W)
    y2, stats2 = pl.pallas_call(
        k2,
        out_shape=(jax.ShapeDtypeStruct((N, D, Cout, Mp), jnp.bfloat16),
                   jax.ShapeDtypeStruct((N, Cout, 2), jnp.float32)),
        grid_spec=pltpu.PrefetchScalarGridSpec(
            num_scalar_prefetch=0,
            grid=(N,),
            in_specs=[
                pl.BlockSpec((None, D, Cout, Mp), lambda n: (n, 0, 0, 0)),
                pl.BlockSpec((Cout, 27 * Cout), lambda n: (0, 0)),
                pl.BlockSpec((1, Mp), lambda n: (0, 0)),
                pl.BlockSpec((Cout, 1), lambda n: (0, 0)),
                pl.BlockSpec((Cout, 1), lambda n: (0, 0)),
            ],
            out_specs=[
                pl.BlockSpec((None, D, Cout, Mp), lambda n: (n, 0, 0, 0)),
                pl.BlockSpec((None, Cout, 2), lambda n: (n, 0, 0)),
            ],
            scratch_shapes=[pltpu.VMEM((4, 9 * Cout, Mp), jnp.bfloat16)],
        ),
        compiler_params=pltpu.CompilerParams(
            dimension_semantics=("parallel",)),
    )(y1, w2m, mask, sc1, sh1)

    sc2, sh2 = _fold_stats(stats2, g2, be2, count)

    kc = functools.partial(_bn_act_kernel, H=H, W=W)
    act = pl.pallas_call(
        kc,
        out_shape=jax.ShapeDtypeStruct((N, Cout, D, H, W), jnp.float32),
        grid_spec=pltpu.PrefetchScalarGridSpec(
            num_scalar_prefetch=0,
            grid=(N, D),
            in_specs=[
                pl.BlockSpec((None, None, Cout, Mp), lambda n, d: (n, d, 0, 0)),
                pl.BlockSpec((Cout, 1), lambda n, d: (0, 0)),
                pl.BlockSpec((Cout, 1), lambda n, d: (0, 0)),
            ],
            out_specs=pl.BlockSpec((None, Cout, None, H, W),
                                   lambda n, d: (n, 0, d, 0, 0)),
        ),
        compiler_params=pltpu.CompilerParams(
            dimension_semantics=("parallel", "parallel")),
    )(y2, sc2, sh2)

    return act


def kernel(x, w1, b1, g1, be1, w2, b2, g2, be2):
    del b1, b2
    return _forward(x.astype(jnp.float32), w1, g1, be1, w2, g2, be2)


# speedup vs baseline: 1.1311x; 1.1311x over previous
import functools

import jax
import jax.numpy as jnp
from jax import lax
from jax.experimental import pallas as pl
from jax.experimental.pallas import tpu as pltpu

_EPS = 1e-5
_SLOPE = 0.1


def _slot_dot(w_ref, im_ref, d, C, Mp):
    r = d % 4
    K = 9 * C
    if r <= 1:
        im = im_ref[r:r + 3].reshape(27 * C, Mp)
        return jnp.dot(w_ref[...], im, preferred_element_type=jnp.float32)
    if r == 2:
        im_a = im_ref[2:4].reshape(18 * C, Mp)
        im_b = im_ref[0]
        ya = jnp.dot(w_ref[:, :2 * K], im_a,
                     preferred_element_type=jnp.float32)
        yb = jnp.dot(w_ref[:, 2 * K:], im_b,
                     preferred_element_type=jnp.float32)
        return ya + yb
    im_a = im_ref[3]
    im_b = im_ref[0:2].reshape(18 * C, Mp)
    ya = jnp.dot(w_ref[:, :K], im_a, preferred_element_type=jnp.float32)
    yb = jnp.dot(w_ref[:, K:], im_b, preferred_element_type=jnp.float32)
    return ya + yb


def _conv1_kernel(x_ref, w_ref, mask_ref, y_ref, stats_ref, im_ref,
                  *, Cin, D, H, W):
    Wp = W + 2
    Mp = H * Wp

    def build(p):
        plane = x_ref[p]
        slot = p % 4
        for t in range(9):
            s = (t // 3) * Wp + (t % 3)
            im_ref[slot, t * Cin:(t + 1) * Cin, :] = plane[:, s:s + Mp]

    build(0)
    build(1)
    ssum = None
    for d in range(D):
        build(d + 2)
        y = _slot_dot(w_ref, im_ref, d, Cin, Mp)
        y_ref[d] = y.astype(y_ref.dtype)
        yv = y * mask_ref[...]
        part = jnp.concatenate(
            [jnp.sum(yv, axis=1, keepdims=True),
             jnp.sum(yv * yv, axis=1, keepdims=True)], axis=1)
        ssum = part if ssum is None else ssum + part

    stats_ref[...] = ssum


def _conv2_kernel(y1_ref, w_ref, mask_ref, sc_ref, sh_ref,
                  y2_ref, stats_ref, im_ref, *, Cout, D, H, W):
    Wp = W + 2
    Mp = H * Wp

    def build(p):
        slot = p % 4
        if p < 1 or p > D:
            im_ref[slot] = jnp.zeros((9 * Cout, Mp), jnp.bfloat16)
            return
        a = y1_ref[p - 1].astype(jnp.float32)
        a = a * sc_ref[...] + sh_ref[...]
        a = jnp.where(a >= 0.0, a, _SLOPE * a)
        a = a * mask_ref[...]
        ab = a.astype(jnp.bfloat16)
        for t in range(9):
            o = Wp + 1 - ((t // 3) * Wp + (t % 3))
            if o > 0:
                z = jnp.zeros((Cout, o), jnp.bfloat16)
                tap = jnp.concatenate([z, ab[:, :Mp - o]], axis=1)
            elif o < 0:
                z = jnp.zeros((Cout, -o), jnp.bfloat16)
                tap = jnp.concatenate([ab[:, -o:], z], axis=1)
            else:
                tap = ab
            im_ref[slot, t * Cout:(t + 1) * Cout, :] = tap

    build(0)
    build(1)
    ssum = None
    for d in range(D):
        build(d + 2)
        y = _slot_dot(w_ref, im_ref, d, Cout, Mp)
        y2_ref[d] = y.astype(y2_ref.dtype)
        yv = y * mask_ref[...]
        part = jnp.concatenate(
            [jnp.sum(yv, axis=1, keepdims=True),
             jnp.sum(yv * yv, axis=1, keepdims=True)], axis=1)
        ssum = part if ssum is None else ssum + part

    stats_ref[...] = ssum


def _bn_act_kernel(y_ref, sc_ref, sh_ref, o_ref):
    z = y_ref[...].astype(jnp.float32) * sc_ref[...] + sh_ref[...]
    o_ref[...] = jnp.where(z >= 0.0, z, _SLOPE * z)


def _fold_stats(stats, gamma, beta, count):
    total = jnp.sum(stats, axis=0)
    mean = total[:, 0] / count
    var = total[:, 1] / count - mean * mean
    scale = gamma * lax.rsqrt(var + _EPS)
    shift = beta - mean * scale
    Cout = gamma.shape[0]
    return scale.reshape(Cout, 1), shift.reshape(Cout, 1)


@jax.jit
def _forward(x, w1, g1, be1, w2, g2, be2):
    N, Cin, D, H, W = x.shape
    Cout = w1.shape[-1]
    Wp = W + 2
    Mp = H * Wp
    Lp = (H + 3) * Wp

    xp = jnp.pad(jnp.transpose(x, (0, 2, 1, 3, 4)),
                 ((0, 0), (1, 1), (0, 0), (1, 2), (1, 1)))
    xf = xp.reshape(N, D + 2, Cin, Lp).astype(jnp.bfloat16)

    w1m = jnp.transpose(w1, (4, 0, 1, 2, 3)).reshape(Cout, 27 * Cin)
    w1m = w1m.astype(jnp.bfloat16)
    w2m = jnp.transpose(w2, (4, 0, 1, 2, 3)).reshape(Cout, 27 * Cout)
    w2m = w2m.astype(jnp.bfloat16)

    mask = ((jnp.arange(Mp, dtype=jnp.int32) % Wp) < W)
    mask = mask.astype(jnp.float32).reshape(1, Mp)

    k1 = functools.partial(_conv1_kernel, Cin=Cin, D=D, H=H, W=W)
    y1, stats1 = pl.pallas_call(
        k1,
        out_shape=(jax.ShapeDtypeStruct((N, D, Cout, Mp), jnp.bfloat16),
                   jax.ShapeDtypeStruct((N, Cout, 2), jnp.float32)),
        grid_spec=pltpu.PrefetchScalarGridSpec(
            num_scalar_prefetch=0,
            grid=(N,),
            in_specs=[
                pl.BlockSpec((None, D + 2, Cin, Lp), lambda n: (n, 0, 0, 0)),
                pl.BlockSpec((Cout, 27 * Cin), lambda n: (0, 0)),
                pl.BlockSpec((1, Mp), lambda n: (0, 0)),
            ],
            out_specs=[
                pl.BlockSpec((None, D, Cout, Mp), lambda n: (n, 0, 0, 0)),
                pl.BlockSpec((None, Cout, 2), lambda n: (n, 0, 0)),
            ],
            scratch_shapes=[pltpu.VMEM((4, 9 * Cin, Mp), jnp.bfloat16)],
        ),
        compiler_params=pltpu.CompilerParams(
            dimension_semantics=("parallel",)),
    )(xf, w1m, mask)

    count = jnp.float32(N * D * H * W)
    sc1, sh1 = _fold_stats(stats1, g1, be1, count)

    k2 = functools.partial(_conv2_kernel, Cout=Cout, D=D, H=H, W=W)
    y2, stats2 = pl.pallas_call(
        k2,
        out_shape=(jax.ShapeDtypeStruct((N, D, Cout, Mp), jnp.bfloat16),
                   jax.ShapeDtypeStruct((N, Cout, 2), jnp.float32)),
        grid_spec=pltpu.PrefetchScalarGridSpec(
            num_scalar_prefetch=0,
            grid=(N,),
            in_specs=[
                pl.BlockSpec((None, D, Cout, Mp), lambda n: (n, 0, 0, 0)),
                pl.BlockSpec((Cout, 27 * Cout), lambda n: (0, 0)),
                pl.BlockSpec((1, Mp), lambda n: (0, 0)),
                pl.BlockSpec((Cout, 1), lambda n: (0, 0)),
                pl.BlockSpec((Cout, 1), lambda n: (0, 0)),
            ],
            out_specs=[
                pl.BlockSpec((None, D, Cout, Mp), lambda n: (n, 0, 0, 0)),
                pl.BlockSpec((None, Cout, 2), lambda n: (n, 0, 0)),
            ],
            scratch_shapes=[pltpu.VMEM((4, 9 * Cout, Mp), jnp.bfloat16)],
        ),
        compiler_params=pltpu.CompilerParams(
            dimension_semantics=("parallel",)),
    )(y1, w2m, mask, sc1, sh1)

    sc2, sh2 = _fold_stats(stats2, g2, be2, count)

    act = pl.pallas_call(
        _bn_act_kernel,
        out_shape=jax.ShapeDtypeStruct((N, D, Cout, Mp), jnp.float32),
        grid_spec=pltpu.PrefetchScalarGridSpec(
            num_scalar_prefetch=0,
            grid=(N,),
            in_specs=[
                pl.BlockSpec((None, D, Cout, Mp), lambda n: (n, 0, 0, 0)),
                pl.BlockSpec((Cout, 1), lambda n: (0, 0)),
                pl.BlockSpec((Cout, 1), lambda n: (0, 0)),
            ],
            out_specs=pl.BlockSpec((None, D, Cout, Mp),
                                   lambda n: (n, 0, 0, 0)),
        ),
        compiler_params=pltpu.CompilerParams(
            dimension_semantics=("parallel",)),
    )(y2, sc2, sh2)

    act = act.reshape(N, D, Cout, H, Wp)[..., :W]
    return jnp.transpose(act, (0, 2, 1, 3, 4))


def kernel(x, w1, b1, g1, be1, w2, b2, g2, be2):
    del b1, b2
    return _forward(x.astype(jnp.float32), w1, g1, be1, w2, g2, be2)
